# baseline (speedup 1.0000x reference)
import functools

import jax
import jax.numpy as jnp
from jax import lax
from jax.experimental import pallas as pl
from jax.experimental.pallas import tpu as pltpu
from jax.experimental.pallas import tpu_sc as plsc

S, D, H, O, E, NS, K = 2048, 1024, 2048, 1024, 8, 1, 2
P = S * K
T = 256
NTE = (P + E * T) // T
PE = NTE * T
NW = 32
NTP = NTE + 8
HS2 = 2
HC2 = H // HS2

_sc_mesh = plsc.VectorSubcoreMesh(core_axis_name="c", subcore_axis_name="s")


def _router_body(x_ref, c_ref, t_ref, logits_ref, idx_ref, wn_ref,
                 wrow_ref, pos_ref, te_ref):
    x = x_ref[...]
    c = c_ref[...]
    xn = x / jnp.clip(jnp.sqrt(jnp.sum(x * x, axis=-1, keepdims=True)), 1e-12, None)
    cn = c / jnp.clip(jnp.sqrt(jnp.sum(c * c, axis=-1, keepdims=True)), 1e-12, None)
    logits = jnp.dot(xn, cn.T, preferred_element_type=jnp.float32) / t_ref[0, 0]
    logits_ref[...] = logits

    m = jnp.max(logits, axis=-1, keepdims=True)
    ex = jnp.exp(logits - m)
    sm = ex / jnp.sum(ex, axis=-1, keepdims=True)

    lane = lax.broadcasted_iota(jnp.int32, (S, E), 1)
    w1 = jnp.max(sm, axis=-1, keepdims=True)
    i1 = jnp.min(jnp.where(sm == w1, lane, E), axis=-1, keepdims=True)
    sm2 = jnp.where(lane == i1, -1.0, sm)
    w2 = jnp.max(sm2, axis=-1, keepdims=True)
    i2 = jnp.min(jnp.where(sm2 == w2, lane, E), axis=-1, keepdims=True)

    idx_ref[...] = jnp.concatenate([i1, i2], axis=1)
    denom = w1 + w2
    wn_ref[...] = jnp.concatenate([w1 / denom, w2 / denom], axis=1)
    wrow_ref[...] = jnp.broadcast_to(
        jnp.concatenate([w1 / denom, w2 / denom], axis=0), (P, 128))

    ind = jnp.concatenate(
        [(lane == i1).astype(jnp.float32).T, (lane == i2).astype(jnp.float32).T],
        axis=1)
    r = ind
    sh = 1
    while sh < P:
        r = r + jnp.concatenate(
            [jnp.zeros((E, sh), jnp.float32), r[:, :P - sh]], axis=1)
        sh *= 2
    counts = r[:, P - 1:P]
    padded = jnp.floor((counts + (T - 1)) * (1.0 / T)) * T
    offp = []
    acc = jnp.zeros((1, 1), jnp.float32)
    for e in range(E):
        offp.append(acc)
        acc = acc + padded[e:e + 1]
    offs = jnp.concatenate(offp, axis=0)
    total = jnp.sum(padded)
    r_pair = jnp.sum(ind * r, axis=0, keepdims=True)
    off_pair = jnp.sum(ind * offs, axis=0, keepdims=True)
    pos_ref[...] = (off_pair + r_pair - 1.0).astype(jnp.int32)

    lane_t = lax.broadcasted_iota(jnp.int32, (1, NTP), 1)
    jt = jnp.minimum(lane_t.astype(jnp.float32) * T, total - T)
    tef = jnp.sum((jt >= offs).astype(jnp.float32), axis=0, keepdims=True) - 1.0
    te_ref[...] = jnp.where(lane_t < NTE, tef, total * (1.0 / T)).astype(jnp.int32)


def _router(xs, centroids, t2):
    return pl.pallas_call(
        _router_body,
        in_specs=[
            pl.BlockSpec((S, D), lambda: (0, 0)),
            pl.BlockSpec((E, D), lambda: (0, 0)),
            pl.BlockSpec(memory_space=pltpu.SMEM),
        ],
        out_specs=[
            pl.BlockSpec((S, E), lambda: (0, 0)),
            pl.BlockSpec((S, K), lambda: (0, 0)),
            pl.BlockSpec((S, K), lambda: (0, 0)),
            pl.BlockSpec((P, 128), lambda: (0, 0)),
            pl.BlockSpec((1, P), lambda: (0, 0)),
            pl.BlockSpec((1, NTP), lambda: (0, 0)),
        ],
        out_shape=[
            jax.ShapeDtypeStruct((S, E), jnp.float32),
            jax.ShapeDtypeStruct((S, K), jnp.int32),
            jax.ShapeDtypeStruct((S, K), jnp.float32),
            jax.ShapeDtypeStruct((P, 128), jnp.float32),
            jax.ShapeDtypeStruct((1, P), jnp.int32),
            jax.ShapeDtypeStruct((1, NTP), jnp.int32),
        ],
    )(xs, centroids, t2)


PPW = P // NW
CC = 32


@functools.partial(
    pl.kernel,
    out_type=[
        jax.ShapeDtypeStruct((PE, D), jnp.float32),
        jax.ShapeDtypeStruct((PE, 128), jnp.float32),
    ],
    mesh=_sc_mesh,
    scratch_types=[
        pltpu.VMEM((2, CC), jnp.int32),
        pltpu.VMEM((2, CC, D), jnp.float32),
        pltpu.VMEM((2, CC, 128), jnp.float32),
        pltpu.SemaphoreType.DMA,
    ],
)
def _dispatch(x_hbm, pos_hbm, wr_hbm, xg_hbm, swr_hbm, pc_v, rows_v, wbuf_v,
              sem):
    wid = lax.axis_index("s") * 2 + lax.axis_index("c")
    base = wid * PPW
    tok0 = base % S

    nch = PPW // CC
    descs = []
    for c in range(nch):
        b = c % 2
        if c >= 2:
            descs[2 * (c - 2)].wait()
            descs[2 * (c - 2) + 1].wait()
        pltpu.sync_copy(pos_hbm.at[pl.ds(base + c * CC, CC)], pc_v.at[b])
        pltpu.sync_copy(x_hbm.at[pl.ds(tok0 + c * CC, CC)], rows_v.at[b])
        pltpu.sync_copy(wr_hbm.at[pl.ds(base + c * CC, CC)], wbuf_v.at[b])
        descs.append(pltpu.async_copy(rows_v.at[b], xg_hbm.at[pc_v.at[b]], sem))
        descs.append(pltpu.async_copy(wbuf_v.at[b], swr_hbm.at[pc_v.at[b]], sem))
    for d in descs[2 * (nch - 2):]:
        d.wait()


def _ffn_body(te_ref, xg_ref, eg_ref, ev_ref, eo_ref, sw_ref, y_ref):
    q = pl.program_id(0)

    @pl.when(q < te_ref[NTE])
    def _():
        xb = xg_ref[...].astype(jnp.bfloat16)
        wg = eg_ref[0].astype(jnp.bfloat16)
        wv = ev_ref[0].astype(jnp.bfloat16)
        wo = eo_ref[0].astype(jnp.bfloat16)
        g = jnp.dot(xb, wg, preferred_element_type=jnp.float32)
        v = jnp.dot(xb, wv, preferred_element_type=jnp.float32)
        prod = (g * jax.nn.sigmoid(g) * v).astype(jnp.bfloat16)
        part = jnp.dot(prod, wo, preferred_element_type=jnp.float32)
        y_ref[...] = part * sw_ref[...][:, 0:1]


def _ffn(te, xg, exp_gate, exp_val, exp_out, swr):
    grid_spec = pltpu.PrefetchScalarGridSpec(
        num_scalar_prefetch=1,
        grid=(NTE,),
        in_specs=[
            pl.BlockSpec((T, D), lambda q, te: (q, 0)),
            pl.BlockSpec((1, D, H), lambda q, te: (te[q], 0, 0)),
            pl.BlockSpec((1, D, H), lambda q, te: (te[q], 0, 0)),
            pl.BlockSpec((1, H, O), lambda q, te: (te[q], 0, 0)),
            pl.BlockSpec((T, 128), lambda q, te: (q, 0)),
        ],
        out_specs=pl.BlockSpec((T, O), lambda q, te: (q, 0)),
    )
    return pl.pallas_call(
        _ffn_body,
        grid_spec=grid_spec,
        out_shape=jax.ShapeDtypeStruct((PE, O), jnp.float32),
        compiler_params=pltpu.CompilerParams(
            dimension_semantics=("arbitrary",)),
    )(te, xg, exp_gate, exp_val, exp_out, swr)


def _shared_body(x_ref, sg_ref, sv_ref, so_ref, out_ref):
    h = pl.program_id(0)

    @pl.when(h == 0)
    def _():
        out_ref[...] = jnp.zeros_like(out_ref)

    xb = x_ref[...]
    wg = sg_ref[...].astype(jnp.bfloat16)
    wv = sv_ref[...].astype(jnp.bfloat16)
    wo = so_ref[...].astype(jnp.bfloat16)
    g = jnp.dot(xb, wg, preferred_element_type=jnp.float32)
    v = jnp.dot(xb, wv, preferred_element_type=jnp.float32)
    prod = (g * jax.nn.sigmoid(g) * v).astype(jnp.bfloat16)
    out_ref[...] += jnp.dot(prod, wo, preferred_element_type=jnp.float32)


def _shared(xb, sh_gate, sh_val, sh_out):
    return pl.pallas_call(
        _shared_body,
        grid=(HS2,),
        in_specs=[
            pl.BlockSpec((S, D), lambda h: (0, 0)),
            pl.BlockSpec((D, HC2), lambda h: (0, h)),
            pl.BlockSpec((D, HC2), lambda h: (0, h)),
            pl.BlockSpec((HC2, NS * O), lambda h: (h, 0)),
        ],
        out_specs=pl.BlockSpec((S, NS * O), lambda h: (0, 0)),
        out_shape=jax.ShapeDtypeStruct((S, NS * O), jnp.float32),
        compiler_params=pltpu.CompilerParams(
            dimension_semantics=("arbitrary",)),
    )(xb, sh_gate, sh_val, sh_out)


TPW = S // NW


@functools.partial(
    pl.kernel,
    out_type=jax.ShapeDtypeStruct((S, O), jnp.float32),
    mesh=_sc_mesh,
    scratch_types=[
        pltpu.VMEM((CC,), jnp.int32),
        pltpu.VMEM((CC,), jnp.int32),
        pltpu.VMEM((CC, O), jnp.float32),
        pltpu.VMEM((CC, O), jnp.float32),
        pltpu.VMEM((CC, O), jnp.float32),
        pltpu.SemaphoreType.DMA,
    ],
)
def _combine(so_hbm, y_hbm, pos_hbm, out_hbm, p0_v, p1_v, acc_v, b0_v, b1_v,
             sem):
    wid = lax.axis_index("s") * 2 + lax.axis_index("c")
    base = wid * TPW

    def chunk_body(c, _):
        t0 = base + c * CC
        pltpu.sync_copy(pos_hbm.at[pl.ds(t0, CC)], p0_v)
        pltpu.sync_copy(pos_hbm.at[pl.ds(S + t0, CC)], p1_v)
        pltpu.sync_copy(so_hbm.at[pl.ds(t0, CC)], acc_v)
        cp0 = pltpu.async_copy(y_hbm.at[p0_v], b0_v, sem)
        cp1 = pltpu.async_copy(y_hbm.at[p1_v], b1_v, sem)
        cp0.wait()
        cp1.wait()
        for j in range(CC):

            def g_body(g, _, j=j):
                for u in range(4):
                    sl = pl.ds(g * 64 + u * 16, 16)
                    acc_v[j, sl] = acc_v[j, sl] + b0_v[j, sl] + b1_v[j, sl]
                return 0

            lax.fori_loop(0, O // 64, g_body, 0)
        pltpu.sync_copy(acc_v, out_hbm.at[pl.ds(t0, CC)])
        return 0

    lax.fori_loop(0, TPW // CC, chunk_body, 0)


def kernel(x, centroids, temperature, exp_gate, exp_val, exp_out, sh_gate,
           sh_val, sh_out):
    xs = x.reshape(S, D)
    t2 = temperature.reshape(1, 1)

    logits, topk_i, wn, wrow, pos2d, te2d = _router(xs, centroids, t2)
    pos = pos2d.reshape(P)
    te = te2d.reshape(NTP)

    xb = xs.astype(jnp.bfloat16)
    xg, swr = _dispatch(xs, pos, wrow)

    so_out = _shared(xb, sh_gate, sh_val, sh_out)
    yw = _ffn(te, xg, exp_gate, exp_val, exp_out, swr)
    out = _combine(so_out, yw, pos)

    return (out.reshape(1, S, O), logits.reshape(1, S, E),
            topk_i.reshape(1, S, K))


# speedup vs baseline: 1.0125x; 1.0125x over previous
import functools

import jax
import jax.numpy as jnp
from jax import lax
from jax.experimental import pallas as pl
from jax.experimental.pallas import tpu as pltpu
from jax.experimental.pallas import tpu_sc as plsc

S, D, H, O, E, NS, K = 2048, 1024, 2048, 1024, 8, 1, 2
P = S * K
T = 256
NTE = (P + E * T) // T
PE = NTE * T
NW = 32
NTP = NTE + 8
HS2 = 2
HC2 = H // HS2

_sc_mesh = plsc.VectorSubcoreMesh(core_axis_name="c", subcore_axis_name="s")


def _router_body(x_ref, c_ref, t_ref, logits_ref, idx_ref, wn_ref,
                 wrow_ref, pos_ref, te_ref):
    x = x_ref[...]
    c = c_ref[...]
    xn = x / jnp.clip(jnp.sqrt(jnp.sum(x * x, axis=-1, keepdims=True)), 1e-12, None)
    cn = c / jnp.clip(jnp.sqrt(jnp.sum(c * c, axis=-1, keepdims=True)), 1e-12, None)
    logits = jnp.dot(xn, cn.T, preferred_element_type=jnp.float32) / t_ref[0, 0]
    logits_ref[...] = logits

    m = jnp.max(logits, axis=-1, keepdims=True)
    ex = jnp.exp(logits - m)
    sm = ex / jnp.sum(ex, axis=-1, keepdims=True)

    lane = lax.broadcasted_iota(jnp.int32, (S, E), 1)
    w1 = jnp.max(sm, axis=-1, keepdims=True)
    i1 = jnp.min(jnp.where(sm == w1, lane, E), axis=-1, keepdims=True)
    sm2 = jnp.where(lane == i1, -1.0, sm)
    w2 = jnp.max(sm2, axis=-1, keepdims=True)
    i2 = jnp.min(jnp.where(sm2 == w2, lane, E), axis=-1, keepdims=True)

    idx_ref[...] = jnp.concatenate([i1, i2], axis=1)
    denom = w1 + w2
    wn_ref[...] = jnp.concatenate([w1 / denom, w2 / denom], axis=1)
    wrow_ref[...] = jnp.broadcast_to(
        jnp.concatenate([w1 / denom, w2 / denom], axis=0), (P, 128))

    ind = jnp.concatenate(
        [(lane == i1).astype(jnp.float32).T, (lane == i2).astype(jnp.float32).T],
        axis=1)
    r = ind
    sh = 1
    while sh < P:
        r = r + jnp.concatenate(
            [jnp.zeros((E, sh), jnp.float32), r[:, :P - sh]], axis=1)
        sh *= 2
    counts = r[:, P - 1:P]
    padded = jnp.floor((counts + (T - 1)) * (1.0 / T)) * T
    offp = []
    acc = jnp.zeros((1, 1), jnp.float32)
    for e in range(E):
        offp.append(acc)
        acc = acc + padded[e:e + 1]
    offs = jnp.concatenate(offp, axis=0)
    total = jnp.sum(padded)
    r_pair = jnp.sum(ind * r, axis=0, keepdims=True)
    off_pair = jnp.sum(ind * offs, axis=0, keepdims=True)
    pos_ref[...] = (off_pair + r_pair - 1.0).astype(jnp.int32)

    lane_t = lax.broadcasted_iota(jnp.int32, (1, NTP), 1)
    jt = jnp.minimum(lane_t.astype(jnp.float32) * T, total - T)
    tef = jnp.sum((jt >= offs).astype(jnp.float32), axis=0, keepdims=True) - 1.0
    te_ref[...] = jnp.where(lane_t < NTE, tef, total * (1.0 / T)).astype(jnp.int32)


def _router(xs, centroids, t2):
    return pl.pallas_call(
        _router_body,
        in_specs=[
            pl.BlockSpec((S, D), lambda: (0, 0)),
            pl.BlockSpec((E, D), lambda: (0, 0)),
            pl.BlockSpec(memory_space=pltpu.SMEM),
        ],
        out_specs=[
            pl.BlockSpec((S, E), lambda: (0, 0)),
            pl.BlockSpec((S, K), lambda: (0, 0)),
            pl.BlockSpec((S, K), lambda: (0, 0)),
            pl.BlockSpec((P, 128), lambda: (0, 0)),
            pl.BlockSpec((1, P), lambda: (0, 0)),
            pl.BlockSpec((1, NTP), lambda: (0, 0)),
        ],
        out_shape=[
            jax.ShapeDtypeStruct((S, E), jnp.float32),
            jax.ShapeDtypeStruct((S, K), jnp.int32),
            jax.ShapeDtypeStruct((S, K), jnp.float32),
            jax.ShapeDtypeStruct((P, 128), jnp.float32),
            jax.ShapeDtypeStruct((1, P), jnp.int32),
            jax.ShapeDtypeStruct((1, NTP), jnp.int32),
        ],
    )(xs, centroids, t2)


PPW = P // NW
CC = 32


@functools.partial(
    pl.kernel,
    out_type=[
        jax.ShapeDtypeStruct((PE, D), jnp.float32),
        jax.ShapeDtypeStruct((PE, 128), jnp.float32),
    ],
    mesh=_sc_mesh,
    scratch_types=[
        pltpu.VMEM((CC,), jnp.int32),
        pltpu.VMEM((CC, D), jnp.float32),
        pltpu.VMEM((CC, 128), jnp.float32),
        pltpu.SemaphoreType.DMA,
    ],
)
def _dispatch(x_hbm, pos_hbm, wr_hbm, xg_hbm, swr_hbm, pc_v, rows_v, wbuf_v,
              sem):
    wid = lax.axis_index("s") * 2 + lax.axis_index("c")
    base = wid * PPW
    tok0 = base % S

    for c in range(PPW // CC):
        pltpu.sync_copy(pos_hbm.at[pl.ds(base + c * CC, CC)], pc_v)
        pltpu.sync_copy(x_hbm.at[pl.ds(tok0 + c * CC, CC)], rows_v)
        pltpu.sync_copy(wr_hbm.at[pl.ds(base + c * CC, CC)], wbuf_v)
        cpx = pltpu.async_copy(rows_v, xg_hbm.at[pc_v], sem)
        cpw = pltpu.async_copy(wbuf_v, swr_hbm.at[pc_v], sem)
        cpx.wait()
        cpw.wait()


def _ffn_body(te_ref, xg_ref, eg_ref, ev_ref, eo_ref, sw_ref, y_ref):
    q = pl.program_id(0)

    @pl.when(q < te_ref[NTE])
    def _():
        xb = xg_ref[...].astype(jnp.bfloat16)
        wg = eg_ref[0].astype(jnp.bfloat16)
        wv = ev_ref[0].astype(jnp.bfloat16)
        wo = eo_ref[0].astype(jnp.bfloat16)
        g = jnp.dot(xb, wg, preferred_element_type=jnp.float32)
        v = jnp.dot(xb, wv, preferred_element_type=jnp.float32)
        prod = (g * jax.nn.sigmoid(g) * v).astype(jnp.bfloat16)
        part = jnp.dot(prod, wo, preferred_element_type=jnp.float32)
        y_ref[...] = part * sw_ref[...][:, 0:1]


def _ffn(te, xg, exp_gate, exp_val, exp_out, swr):
    grid_spec = pltpu.PrefetchScalarGridSpec(
        num_scalar_prefetch=1,
        grid=(NTE,),
        in_specs=[
            pl.BlockSpec((T, D), lambda q, te: (q, 0)),
            pl.BlockSpec((1, D, H), lambda q, te: (te[q], 0, 0)),
            pl.BlockSpec((1, D, H), lambda q, te: (te[q], 0, 0)),
            pl.BlockSpec((1, H, O), lambda q, te: (te[q], 0, 0)),
            pl.BlockSpec((T, 128), lambda q, te: (q, 0)),
        ],
        out_specs=pl.BlockSpec((T, O), lambda q, te: (q, 0)),
    )
    return pl.pallas_call(
        _ffn_body,
        grid_spec=grid_spec,
        out_shape=jax.ShapeDtypeStruct((PE, O), jnp.float32),
        compiler_params=pltpu.CompilerParams(
            dimension_semantics=("arbitrary",)),
    )(te, xg, exp_gate, exp_val, exp_out, swr)


def _shared_body(x_ref, sg_ref, sv_ref, so_ref, out_ref):
    h = pl.program_id(0)

    @pl.when(h == 0)
    def _():
        out_ref[...] = jnp.zeros_like(out_ref)

    xb = x_ref[...]
    wg = sg_ref[...].astype(jnp.bfloat16)
    wv = sv_ref[...].astype(jnp.bfloat16)
    wo = so_ref[...].astype(jnp.bfloat16)
    g = jnp.dot(xb, wg, preferred_element_type=jnp.float32)
    v = jnp.dot(xb, wv, preferred_element_type=jnp.float32)
    prod = (g * jax.nn.sigmoid(g) * v).astype(jnp.bfloat16)
    out_ref[...] += jnp.dot(prod, wo, preferred_element_type=jnp.float32)


def _shared(xb, sh_gate, sh_val, sh_out):
    return pl.pallas_call(
        _shared_body,
        grid=(HS2,),
        in_specs=[
            pl.BlockSpec((S, D), lambda h: (0, 0)),
            pl.BlockSpec((D, HC2), lambda h: (0, h)),
            pl.BlockSpec((D, HC2), lambda h: (0, h)),
            pl.BlockSpec((HC2, NS * O), lambda h: (h, 0)),
        ],
        out_specs=pl.BlockSpec((S, NS * O), lambda h: (0, 0)),
        out_shape=jax.ShapeDtypeStruct((S, NS * O), jnp.float32),
        compiler_params=pltpu.CompilerParams(
            dimension_semantics=("arbitrary",)),
    )(xb, sh_gate, sh_val, sh_out)


TPW = S // NW


@functools.partial(
    pl.kernel,
    out_type=jax.ShapeDtypeStruct((S, O), jnp.float32),
    mesh=_sc_mesh,
    scratch_types=[
        pltpu.VMEM((CC,), jnp.int32),
        pltpu.VMEM((CC,), jnp.int32),
        pltpu.VMEM((CC, O), jnp.float32),
        pltpu.VMEM((CC, O), jnp.float32),
        pltpu.VMEM((CC, O), jnp.float32),
        pltpu.SemaphoreType.DMA,
    ],
)
def _combine(so_hbm, y_hbm, pos_hbm, out_hbm, p0_v, p1_v, acc_v, b0_v, b1_v,
             sem):
    wid = lax.axis_index("s") * 2 + lax.axis_index("c")
    base = wid * TPW

    def chunk_body(c, _):
        t0 = base + c * CC
        pltpu.sync_copy(pos_hbm.at[pl.ds(t0, CC)], p0_v)
        pltpu.sync_copy(pos_hbm.at[pl.ds(S + t0, CC)], p1_v)
        pltpu.sync_copy(so_hbm.at[pl.ds(t0, CC)], acc_v)
        cp0 = pltpu.async_copy(y_hbm.at[p0_v], b0_v, sem)
        cp1 = pltpu.async_copy(y_hbm.at[p1_v], b1_v, sem)
        cp0.wait()
        cp1.wait()
        for j in range(CC):

            def g_body(g, _, j=j):
                for u in range(4):
                    sl = pl.ds(g * 64 + u * 16, 16)
                    acc_v[j, sl] = acc_v[j, sl] + b0_v[j, sl] + b1_v[j, sl]
                return 0

            lax.fori_loop(0, O // 64, g_body, 0)
        pltpu.sync_copy(acc_v, out_hbm.at[pl.ds(t0, CC)])
        return 0

    lax.fori_loop(0, TPW // CC, chunk_body, 0)


def kernel(x, centroids, temperature, exp_gate, exp_val, exp_out, sh_gate,
           sh_val, sh_out):
    xs = x.reshape(S, D)
    t2 = temperature.reshape(1, 1)

    logits, topk_i, wn, wrow, pos2d, te2d = _router(xs, centroids, t2)
    pos = pos2d.reshape(P)
    te = te2d.reshape(NTP)

    xb = xs.astype(jnp.bfloat16)
    xg, swr = _dispatch(xs, pos, wrow)

    so_out = _shared(xb, sh_gate, sh_val, sh_out)
    yw = _ffn(te, xg, exp_gate, exp_val, exp_out, swr)
    out = _combine(so_out, yw, pos)

    return (out.reshape(1, S, O), logits.reshape(1, S, E),
            topk_i.reshape(1, S, K))
